# Initial kernel scaffold; baseline (speedup 1.0000x reference)
import jax
import jax.numpy as jnp
from jax.experimental import pallas as pl


def kernel(x, edge_index, edge_weight, batch, W1, b1, W2, b2, W3, b3, Wr, br, Wc, bc):
    raise NotImplementedError("write your pallas kernel here")


# speedup vs baseline: 9.3076x; 9.3076x over previous
import functools

import jax
import jax.numpy as jnp
from jax import lax
from jax.experimental import pallas as pl
from jax.experimental.pallas import tpu as pltpu
from jax.experimental.pallas import tpu_sc as plsc

N = 10000
E = 320000
D = 128
G = 8
NP = 10240
NC = 2
NS = 16
NW = NC * NS
K = 128
CPW = 79
EPW = CPW * K
EP = NW * EPW
RPS = NP // NS
BN = 1024

_sc_mesh = plsc.VectorSubcoreMesh(core_axis_name="c", subcore_axis_name="s")
_sc_params = pltpu.CompilerParams(needs_layout_passes=False)


@functools.partial(
    pl.kernel,
    out_type=jax.ShapeDtypeStruct((NW, NP), jnp.float32),
    mesh=_sc_mesh,
    scratch_types=[
        pltpu.VMEM((EPW,), jnp.int32),
        pltpu.VMEM((EPW,), jnp.float32),
        pltpu.VMEM((NP,), jnp.float32),
    ],
    compiler_params=_sc_params,
)
def _deg_kernel(col_hbm, ew_hbm, znp_hbm, out_hbm, col_v, ew_v, deg_v):
    c = lax.axis_index("c")
    s = lax.axis_index("s")
    wid = c * NS + s
    pltpu.sync_copy(col_hbm.at[wid], col_v)
    pltpu.sync_copy(ew_hbm.at[wid], ew_v)
    pltpu.sync_copy(znp_hbm, deg_v)

    def body(i, carry):
        idx = col_v[pl.ds(i * 16, 16)]
        w = ew_v[pl.ds(i * 16, 16)]
        plsc.addupdate_scatter(deg_v, [idx], w)
        return carry

    lax.fori_loop(0, EPW // 16, body, 0)
    pltpu.sync_copy(deg_v, out_hbm.at[wid])


@functools.partial(
    pl.kernel,
    out_type=jax.ShapeDtypeStruct((NC, NP, D), jnp.float32),
    mesh=_sc_mesh,
    scratch_types=[
        pltpu.VMEM((CPW, K), jnp.int32),
        pltpu.VMEM((CPW, K), jnp.int32),
        pltpu.VMEM((EPW,), jnp.float32),
        pltpu.VMEM((K, D), jnp.float32),
        pltpu.VMEM_SHARED((NP, D), jnp.float32),
        pltpu.SemaphoreType.DMA,
    ],
    compiler_params=_sc_params,
)
def _edge_scatter_kernel(y_hbm, row_hbm, col_hbm, ew_hbm, zr_hbm, out_hbm,
                         row_v, col_v, ew_v, buf, acc, sem):
    c = lax.axis_index("c")
    s = lax.axis_index("s")
    wid = c * NS + s
    pltpu.sync_copy(row_hbm.at[wid], row_v)
    pltpu.sync_copy(col_hbm.at[wid], col_v)
    pltpu.sync_copy(ew_hbm.at[wid], ew_v)
    pltpu.sync_copy(zr_hbm, acc.at[pl.ds(s * RPS, RPS)])
    plsc.subcore_barrier()

    def chunk(g, carry):
        pltpu.async_copy(y_hbm.at[row_v.at[g]], buf, sem).wait()

        def scale16(g2, inner):
            base = g2 * 16
            wv = ew_v[pl.ds(g * K + base, 16)]
            for lane in range(16):
                w = wv[lane]
                j = base + lane
                for i in range(D // 16):
                    sl = pl.ds(i * 16, 16)
                    buf[j, sl] = buf[j, sl] * w
            return inner

        lax.fori_loop(0, K // 16, scale16, 0)
        pltpu.sync_copy(buf, acc.at[col_v.at[g]], add=True)
        return carry

    lax.fori_loop(0, CPW, chunk, 0)
    plsc.subcore_barrier()
    for k in range(RPS // K):
        base = s * RPS + k * K
        pltpu.sync_copy(acc.at[pl.ds(base, K)], buf)
        pltpu.sync_copy(buf, out_hbm.at[c, pl.ds(base, K)])


def _dinv_body(parts_ref, o_ref):
    deg = jnp.sum(parts_ref[...], axis=1, keepdims=True) + 1.0
    o_ref[...] = jnp.broadcast_to(lax.rsqrt(deg), (NP, D))


def _mm1_body(x_ref, w_ref, dinvb_ref, xw_ref, y_ref):
    xw = jnp.dot(x_ref[...], w_ref[...], preferred_element_type=jnp.float32)
    xw_ref[...] = xw
    y_ref[...] = xw * dinvb_ref[...]


def _layer_body(a0_ref, a1_ref, xw_ref, dinvb_ref, b_ref, w_ref,
                xwn_ref, yn_ref):
    dinv = dinvb_ref[...]
    h = dinv * (a0_ref[...] + a1_ref[...]) + dinv * dinv * xw_ref[...] + b_ref[...]
    h = jnp.maximum(h, 0.0)
    xwn = jnp.dot(h, w_ref[...], preferred_element_type=jnp.float32)
    xwn_ref[...] = xwn
    yn_ref[...] = xwn * dinv


def _pool_body(a0_ref, a1_ref, xw_ref, dinvb_ref, b_ref, bat_ref,
               sums_ref, cnts_ref):
    dinv = dinvb_ref[...]
    h3 = dinv * (a0_ref[...] + a1_ref[...]) + dinv * dinv * xw_ref[...] + b_ref[...]
    onehot = (bat_ref[...] ==
              lax.broadcasted_iota(jnp.int32, (BN, D), 1)).astype(jnp.float32)
    dn = (((0,), (0,)), ((), ()))
    psum = lax.dot_general(onehot, h3, dn, preferred_element_type=jnp.float32)
    pcnt = lax.dot_general(onehot, jnp.ones_like(h3), dn,
                           preferred_element_type=jnp.float32)

    @pl.when(pl.program_id(0) == 0)
    def _():
        sums_ref[...] = psum
        cnts_ref[...] = pcnt

    @pl.when(pl.program_id(0) != 0)
    def _():
        sums_ref[...] += psum
        cnts_ref[...] += pcnt


def _head_body(sums_ref, cnts_ref, w_ref, b_ref, o_ref):
    pooled = sums_ref[...] / jnp.maximum(cnts_ref[...], 1.0)
    p8 = pooled[0:G, :]
    o_ref[...] = jnp.dot(p8, w_ref[...],
                         preferred_element_type=jnp.float32) + b_ref[...]


def _row_spec():
    return pl.BlockSpec((BN, D), lambda i: (i, 0))


def _full_spec(shape):
    return pl.BlockSpec(shape, lambda i: tuple(0 for _ in shape))


def kernel(x, edge_index, edge_weight, batch,
           W1, b1, W2, b2, W3, b3, Wr, br, Wc, bc):
    f32 = jnp.float32
    row = edge_index[0]
    col = edge_index[1]

    pad_e = EP - E
    row3 = jnp.concatenate([row, jnp.zeros((pad_e,), row.dtype)]).reshape(NW, CPW, K)
    col3 = jnp.concatenate([col, jnp.zeros((pad_e,), col.dtype)]).reshape(NW, CPW, K)
    ewf = jnp.concatenate([edge_weight, jnp.zeros((pad_e,), f32)]).reshape(NW, EPW)
    colf = col3.reshape(NW, EPW)
    x_p = jnp.concatenate([x, jnp.zeros((NP - N, D), f32)])
    z_np = jnp.zeros((NP,), f32)
    z_rows = jnp.zeros((RPS, D), f32)
    batch_p = jnp.concatenate([batch, jnp.full((NP - N,), G, batch.dtype)])
    batchb = jnp.broadcast_to(batch_p.astype(jnp.int32)[:, None], (NP, D))
    b1r = b1.reshape(1, D)
    b2r = b2.reshape(1, D)
    b3r = b3.reshape(1, D)
    w_head = jnp.zeros((D, D), f32).at[:, 0:3].set(Wr).at[:, 3:5].set(Wc)
    b_head = jnp.zeros((1, D), f32).at[0, 0:3].set(br).at[0, 3:5].set(bc)

    nb = NP // BN

    deg_parts = _deg_kernel(colf, ewf, z_np)
    dinvb = pl.pallas_call(
        _dinv_body,
        out_shape=jax.ShapeDtypeStruct((NP, D), f32),
        grid=(1,),
        in_specs=[_full_spec((NP, NW))],
        out_specs=_full_spec((NP, D)),
    )(deg_parts.T)

    xw1, y1 = pl.pallas_call(
        _mm1_body,
        out_shape=(jax.ShapeDtypeStruct((NP, D), f32),
                   jax.ShapeDtypeStruct((NP, D), f32)),
        grid=(nb,),
        in_specs=[_row_spec(), _full_spec((D, D)), _row_spec()],
        out_specs=(_row_spec(), _row_spec()),
    )(x_p, W1, dinvb)

    def tc_layer(acc, xw, b_r, w_next):
        return pl.pallas_call(
            _layer_body,
            out_shape=(jax.ShapeDtypeStruct((NP, D), f32),
                       jax.ShapeDtypeStruct((NP, D), f32)),
            grid=(nb,),
            in_specs=[_row_spec(), _row_spec(), _row_spec(), _row_spec(),
                      _full_spec((1, D)), _full_spec((D, D))],
            out_specs=(_row_spec(), _row_spec()),
        )(acc[0], acc[1], xw, dinvb, b_r, w_next)

    acc1 = _edge_scatter_kernel(y1, row3, col3, ewf, z_rows)
    xw2, y2 = tc_layer(acc1, xw1, b1r, W2)
    acc2 = _edge_scatter_kernel(y2, row3, col3, ewf, z_rows)
    xw3, y3 = tc_layer(acc2, xw2, b2r, W3)
    acc3 = _edge_scatter_kernel(y3, row3, col3, ewf, z_rows)

    sums, cnts = pl.pallas_call(
        _pool_body,
        out_shape=(jax.ShapeDtypeStruct((D, D), f32),
                   jax.ShapeDtypeStruct((D, D), f32)),
        grid=(nb,),
        in_specs=[_row_spec(), _row_spec(), _row_spec(), _row_spec(),
                  _full_spec((1, D)), _row_spec()],
        out_specs=(_full_spec((D, D)), _full_spec((D, D))),
    )(acc3[0], acc3[1], xw3, dinvb, b3r, batchb)

    out = pl.pallas_call(
        _head_body,
        out_shape=jax.ShapeDtypeStruct((G, D), f32),
        grid=(1,),
        in_specs=[_full_spec((D, D)), _full_spec((D, D)),
                  _full_spec((D, D)), _full_spec((1, D))],
        out_specs=_full_spec((G, D)),
    )(sums, cnts, w_head, b_head)

    return out[:, 0:3], out[:, 3:5]
